# Initial kernel scaffold; baseline (speedup 1.0000x reference)
import jax
import jax.numpy as jnp
from jax.experimental import pallas as pl


def kernel(x, token_emb, pos_emb):
    raise NotImplementedError("write your pallas kernel here")


# speedup vs baseline: 1.3692x; 1.3692x over previous
import functools

import jax
import jax.numpy as jnp
from jax import lax
from jax.experimental import pallas as pl
from jax.experimental.pallas import tpu as pltpu
from jax.experimental.pallas import tpu_sc as plsc

B = 4
S = 2048
D = 768
LANES = 16
D_VECS = D // LANES

NUM_CORES = 2
NUM_SUBCORES = 16
NW = NUM_CORES * NUM_SUBCORES
ROWS_PER_W = (B * S) // NW
CHUNK = 32
NCHUNK = ROWS_PER_W // CHUNK


def _make_kernel():
    mesh = plsc.VectorSubcoreMesh(core_axis_name="c", subcore_axis_name="s")

    @functools.partial(
        pl.kernel,
        mesh=mesh,
        out_type=jax.ShapeDtypeStruct((B * S, D), jnp.float32),
        scratch_types=[
            pltpu.VMEM((ROWS_PER_W,), jnp.int32),
            pltpu.VMEM((CHUNK, D), jnp.float32),
            pltpu.VMEM((CHUNK, D), jnp.float32),
            pltpu.SemaphoreType.DMA,
        ],
    )
    def emb_kernel(x_hbm, tok_hbm, pos_hbm, out_hbm, idx_v, tok_v, pos_v, sem):
        wid = lax.axis_index("s") * NUM_CORES + lax.axis_index("c")
        base = wid * ROWS_PER_W
        s_base = base % S

        pltpu.sync_copy(x_hbm.at[pl.ds(base, ROWS_PER_W)], idx_v)

        def chunk_body(ci, carry):
            off = ci * CHUNK
            gather = pltpu.async_copy(
                tok_hbm.at[idx_v.at[pl.ds(off, CHUNK)]], tok_v, sem
            )
            pltpu.sync_copy(pos_hbm.at[pl.ds(s_base + off, CHUNK)], pos_v)
            gather.wait()

            def add_row(r, c2):
                for c in range(D_VECS):
                    sl = pl.ds(c * LANES, LANES)
                    tok_v[r, sl] = tok_v[r, sl] + pos_v[r, sl]
                return c2

            lax.fori_loop(0, CHUNK, add_row, 0)
            pltpu.sync_copy(tok_v, out_hbm.at[pl.ds(base + off, CHUNK)])
            return carry

        lax.fori_loop(0, NCHUNK, chunk_body, 0)

    return emb_kernel


_emb_kernel = _make_kernel()


def kernel(x, token_emb, pos_emb):
    x_flat = x.reshape(-1).astype(jnp.int32)
    out = _emb_kernel(x_flat, token_emb, pos_emb)
    return out.reshape(B, S, D)


# speedup vs baseline: 1.7104x; 1.2492x over previous
import functools

import jax
import jax.numpy as jnp
from jax import lax
from jax.experimental import pallas as pl
from jax.experimental.pallas import tpu as pltpu
from jax.experimental.pallas import tpu_sc as plsc

B = 4
S = 2048
D = 768
LANES = 16
D_VECS = D // LANES

NUM_CORES = 2
NUM_SUBCORES = 16
NW = NUM_CORES * NUM_SUBCORES
ROWS_PER_W = (B * S) // NW
CHUNK = 32
NCHUNK = ROWS_PER_W // CHUNK


def _make_kernel():
    mesh = plsc.VectorSubcoreMesh(core_axis_name="c", subcore_axis_name="s")

    @functools.partial(
        pl.kernel,
        mesh=mesh,
        out_type=jax.ShapeDtypeStruct((B * S, D), jnp.float32),
        scratch_types=[
            pltpu.VMEM((ROWS_PER_W,), jnp.int32),
            pltpu.VMEM((CHUNK, D), jnp.float32),
            pltpu.VMEM((CHUNK, D), jnp.float32),
            pltpu.VMEM((CHUNK, D), jnp.float32),
            pltpu.VMEM((CHUNK, D), jnp.float32),
            pltpu.SemaphoreType.DMA,
            pltpu.SemaphoreType.DMA,
            pltpu.SemaphoreType.DMA,
            pltpu.SemaphoreType.DMA,
            pltpu.SemaphoreType.DMA,
            pltpu.SemaphoreType.DMA,
        ],
    )
    def emb_kernel(x_hbm, tok_hbm, pos_hbm, out_hbm,
                   idx_v, t0, t1, p0, p1,
                   gs0, gs1, ps0, ps1, os0, os1):
        wid = lax.axis_index("s") * NUM_CORES + lax.axis_index("c")
        base = wid * ROWS_PER_W
        s_base = base % S

        toks = (t0, t1)
        poss = (p0, p1)
        gss = (gs0, gs1)
        pss = (ps0, ps1)
        oss = (os0, os1)

        pltpu.sync_copy(x_hbm.at[pl.ds(base, ROWS_PER_W)], idx_v)

        def start_chunk(ci, b):
            off = ci * CHUNK
            pltpu.async_copy(tok_hbm.at[idx_v.at[pl.ds(off, CHUNK)]],
                             toks[b], gss[b])
            pltpu.async_copy(pos_hbm.at[pl.ds(s_base + off, CHUNK)],
                             poss[b], pss[b])

        def process_chunk(ci, b):
            off = ci * CHUNK
            pltpu.make_async_copy(tok_hbm.at[idx_v.at[pl.ds(off, CHUNK)]],
                                  toks[b], gss[b]).wait()
            pltpu.make_async_copy(pos_hbm.at[pl.ds(s_base + off, CHUNK)],
                                  poss[b], pss[b]).wait()

            def add_row(r, c2):
                for c in range(D_VECS):
                    sl = pl.ds(c * LANES, LANES)
                    toks[b][r, sl] = toks[b][r, sl] + poss[b][r, sl]
                return c2

            lax.fori_loop(0, CHUNK, add_row, 0)
            pltpu.async_copy(toks[b], out_hbm.at[pl.ds(base + off, CHUNK)],
                             oss[b])

        def wait_out(b):
            pltpu.make_async_copy(toks[b], out_hbm.at[pl.ds(base, CHUNK)],
                                  oss[b]).wait()

        start_chunk(0, 0)

        def outer(i2, carry):
            for b in (0, 1):
                ci = i2 * 2 + b
                nb = b ^ 1

                @pl.when(ci + 1 < NCHUNK)
                def _():
                    @pl.when(ci >= 1)
                    def _():
                        wait_out(nb)
                    start_chunk(ci + 1, nb)

                process_chunk(ci, b)
            return carry

        lax.fori_loop(0, NCHUNK // 2, outer, 0)
        wait_out(0)
        wait_out(1)

    return emb_kernel


_emb_kernel = _make_kernel()


def kernel(x, token_emb, pos_emb):
    x_flat = x.reshape(-1).astype(jnp.int32)
    out = _emb_kernel(x_flat, token_emb, pos_emb)
    return out.reshape(B, S, D)
